# baseline (speedup 1.0000x reference)
import jax
import jax.numpy as jnp
from jax.experimental import pallas as pl
from jax.experimental.pallas import tpu as pltpu

B, H, S, D, Q = 8, 8, 2048, 128, 1


def _update_body(pos_ref, k_ref, v_ref, nk_ref, nv_ref, ko_ref, vo_ref,
                 bulk_sem, krow_sem, vrow_sem):
    bulk_k = pltpu.make_async_copy(k_ref, ko_ref, bulk_sem.at[0])
    bulk_v = pltpu.make_async_copy(v_ref, vo_ref, bulk_sem.at[1])
    bulk_k.start()
    bulk_v.start()

    bulk_k.wait()
    k_rows = []
    for b in range(B):
        p = pos_ref[b]
        c = pltpu.make_async_copy(
            nk_ref.at[b], ko_ref.at[b, :, pl.ds(p, 1), :], krow_sem.at[b])
        c.start()
        k_rows.append(c)

    bulk_v.wait()
    v_rows = []
    for b in range(B):
        p = pos_ref[b]
        c = pltpu.make_async_copy(
            nv_ref.at[b], vo_ref.at[b, :, pl.ds(p, 1), :], vrow_sem.at[b])
        c.start()
        v_rows.append(c)

    for c in k_rows:
        c.wait()
    for c in v_rows:
        c.wait()


@jax.jit
def kernel(k_cache, v_cache, new_k, new_v, position_ids):
    pos = position_ids.reshape(B)

    hbm = pl.BlockSpec(memory_space=pl.ANY)
    k_out, v_out = pl.pallas_call(
        _update_body,
        in_specs=[
            pl.BlockSpec(memory_space=pltpu.MemorySpace.SMEM),
            hbm, hbm, hbm, hbm,
        ],
        out_specs=[hbm, hbm],
        out_shape=[
            jax.ShapeDtypeStruct((B, H, S, D), k_cache.dtype),
            jax.ShapeDtypeStruct((B, H, S, D), v_cache.dtype),
        ],
        scratch_shapes=[
            pltpu.SemaphoreType.DMA((2,)),
            pltpu.SemaphoreType.DMA((B,)),
            pltpu.SemaphoreType.DMA((B,)),
        ],
    )(pos, k_cache, v_cache, new_k, new_v)
    return (k_out, v_out)


# speedup vs baseline: 30.5692x; 30.5692x over previous
import jax
import jax.numpy as jnp
from jax.experimental import pallas as pl
from jax.experimental.pallas import tpu as pltpu

B, H, S, D, Q = 8, 8, 2048, 128, 1
SB = 1024


def _update_body(pos_ref, k_ref, v_ref, nk_ref, nv_ref, ko_ref, vo_ref):
    b = pl.program_id(0)
    j = pl.program_id(2)
    base = j * SB
    p = pos_ref[b]
    ko_ref[...] = k_ref[...]
    vo_ref[...] = v_ref[...]

    @pl.when(jnp.logical_and(p >= base, p < base + SB))
    def _():
        ko_ref[0, 0, p - base, :] = nk_ref[0, 0, 0, :]
        vo_ref[0, 0, p - base, :] = nv_ref[0, 0, 0, :]


@jax.jit
def kernel(k_cache, v_cache, new_k, new_v, position_ids):
    pos = position_ids.reshape(B)

    cache_spec = pl.BlockSpec((1, 1, SB, D), lambda b, h, j, pos_ref: (b, h, j, 0))
    new_spec = pl.BlockSpec((1, 1, Q, D), lambda b, h, j, pos_ref: (b, h, 0, 0))

    grid_spec = pltpu.PrefetchScalarGridSpec(
        num_scalar_prefetch=1,
        grid=(B, H, S // SB),
        in_specs=[cache_spec, cache_spec, new_spec, new_spec],
        out_specs=[cache_spec, cache_spec],
    )

    k_out, v_out = pl.pallas_call(
        _update_body,
        grid_spec=grid_spec,
        out_shape=[
            jax.ShapeDtypeStruct((B, H, S, D), k_cache.dtype),
            jax.ShapeDtypeStruct((B, H, S, D), v_cache.dtype),
        ],
    )(pos, k_cache, v_cache, new_k, new_v)
    return (k_out, v_out)


# speedup vs baseline: 47.5119x; 1.5542x over previous
import jax
import jax.numpy as jnp
from jax.experimental import pallas as pl
from jax.experimental.pallas import tpu as pltpu

B, H, S, D, Q = 8, 8, 2048, 128, 1
HB = 2


def _update_body(pos_ref, k_ref, v_ref, nk_ref, nv_ref, ko_ref, vo_ref):
    b = pl.program_id(0)
    p = pos_ref[b]
    ko_ref[...] = k_ref[...]
    vo_ref[...] = v_ref[...]
    ko_ref[0, :, p, :] = nk_ref[0, :, 0, :]
    vo_ref[0, :, p, :] = nv_ref[0, :, 0, :]


@jax.jit
def kernel(k_cache, v_cache, new_k, new_v, position_ids):
    pos = position_ids.reshape(B)

    cache_spec = pl.BlockSpec((1, HB, S, D), lambda b, h, pos_ref: (b, h, 0, 0))
    new_spec = pl.BlockSpec((1, HB, Q, D), lambda b, h, pos_ref: (b, h, 0, 0))

    grid_spec = pltpu.PrefetchScalarGridSpec(
        num_scalar_prefetch=1,
        grid=(B, H // HB),
        in_specs=[cache_spec, cache_spec, new_spec, new_spec],
        out_specs=[cache_spec, cache_spec],
    )

    k_out, v_out = pl.pallas_call(
        _update_body,
        grid_spec=grid_spec,
        out_shape=[
            jax.ShapeDtypeStruct((B, H, S, D), k_cache.dtype),
            jax.ShapeDtypeStruct((B, H, S, D), v_cache.dtype),
        ],
    )(pos, k_cache, v_cache, new_k, new_v)
    return (k_out, v_out)


# speedup vs baseline: 48.4276x; 1.0193x over previous
import jax
import jax.numpy as jnp
from jax.experimental import pallas as pl
from jax.experimental.pallas import tpu as pltpu

B, H, S, D, Q = 8, 8, 2048, 128, 1
HB = 4


def _update_body(pos_ref, k_ref, v_ref, nk_ref, nv_ref, ko_ref, vo_ref):
    b = pl.program_id(0)
    p = pos_ref[b]
    ko_ref[...] = k_ref[...]
    vo_ref[...] = v_ref[...]
    ko_ref[0, :, p, :] = nk_ref[0, :, 0, :]
    vo_ref[0, :, p, :] = nv_ref[0, :, 0, :]


@jax.jit
def kernel(k_cache, v_cache, new_k, new_v, position_ids):
    pos = position_ids.reshape(B)

    cache_spec = pl.BlockSpec((1, HB, S, D), lambda b, h, pos_ref: (b, h, 0, 0))
    new_spec = pl.BlockSpec((1, HB, Q, D), lambda b, h, pos_ref: (b, h, 0, 0))

    grid_spec = pltpu.PrefetchScalarGridSpec(
        num_scalar_prefetch=1,
        grid=(B, H // HB),
        in_specs=[cache_spec, cache_spec, new_spec, new_spec],
        out_specs=[cache_spec, cache_spec],
    )

    k_out, v_out = pl.pallas_call(
        _update_body,
        grid_spec=grid_spec,
        out_shape=[
            jax.ShapeDtypeStruct((B, H, S, D), k_cache.dtype),
            jax.ShapeDtypeStruct((B, H, S, D), v_cache.dtype),
        ],
    )(pos, k_cache, v_cache, new_k, new_v)
    return (k_out, v_out)
